# Initial kernel scaffold; baseline (speedup 1.0000x reference)
import functools

import jax
import jax.numpy as jnp
from jax import lax
from jax.experimental import pallas as pl
from jax.experimental.pallas import tpu as pltpu
from jax.experimental.pallas import tpu_sc as plsc

NC = 2
NS = 16
NW = NC * NS
LANES = 16

CHUNK = 80


def _sc_aggregate(src, dst, x, edge_weight, *, n_chunks):
    n, d = x.shape
    rows_per_tile = n // NS
    zrows = 125
    n_stage = rows_per_tile // zrows

    mesh = plsc.VectorSubcoreMesh(
        core_axis_name="c", subcore_axis_name="s", num_cores=NC, num_subcores=NS
    )

    @functools.partial(
        pl.kernel,
        out_type=jax.ShapeDtypeStruct((NC, n, d), jnp.float32),
        mesh=mesh,
        scratch_types=[
            pltpu.VMEM((n_chunks, CHUNK), jnp.int32),
            pltpu.VMEM((n_chunks, CHUNK), jnp.int32),
            pltpu.VMEM((CHUNK, d), jnp.float32),
            pltpu.VMEM((CHUNK, d), jnp.float32),
            pltpu.VMEM((125, d), jnp.float32),
            pltpu.VMEM_SHARED((n, d), jnp.float32),
            pltpu.SemaphoreType.DMA,
        ],
    )
    def agg_kernel(src_hbm, dst_hbm, x_hbm, ew_hbm, out_hbm,
                   src_v, dst_v, xbuf, ewbuf, stage, agg_sh, sem):
        zrows_ = 125
        c = lax.axis_index("c")
        s = lax.axis_index("s")
        wid = s * NC + c
        edge_base = wid * (n_chunks * CHUNK)

        pltpu.sync_copy(src_hbm.at[wid], src_v)
        pltpu.sync_copy(dst_hbm.at[wid], dst_v)

        zero = jnp.zeros((LANES,), jnp.float32)

        def zero_row(r, _):
            for cc in range(d // LANES):
                stage[r, pl.ds(cc * LANES, LANES)] = zero
            return 0

        lax.fori_loop(0, zrows_, zero_row, 0)
        for k in range(n_stage):
            pltpu.sync_copy(
                stage, agg_sh.at[pl.ds(s * rows_per_tile + k * zrows_, zrows_)]
            )
        plsc.subcore_barrier()

        def chunk_body(j, _):
            pltpu.sync_copy(ew_hbm.at[pl.ds(edge_base + j * CHUNK, CHUNK)], ewbuf)
            pltpu.async_copy(x_hbm.at[src_v.at[j]], xbuf, sem).wait()

            def mul_row(r, _):
                for cc in range(d // LANES):
                    sl = pl.ds(cc * LANES, LANES)
                    ewbuf[r, sl] = ewbuf[r, sl] * xbuf[r, sl]
                return 0

            lax.fori_loop(0, CHUNK, mul_row, 0)
            pltpu.sync_copy(ewbuf, agg_sh.at[dst_v.at[j]], add=True)
            return 0

        lax.fori_loop(0, n_chunks, chunk_body, 0)
        plsc.subcore_barrier()

        for k in range(n_stage):
            rows = pl.ds(s * rows_per_tile + k * zrows_, zrows_)
            pltpu.sync_copy(agg_sh.at[rows], stage)
            pltpu.sync_copy(stage, out_hbm.at[c].at[rows])

    return agg_kernel(src, dst, x, edge_weight)


def _tc_linear(partials, x, w_l, w_r, b_l, *, block_rows=400):
    n, d = x.shape

    def body(p_ref, x_ref, wl_ref, wr_ref, b_ref, o_ref):
        a = p_ref[0] + p_ref[1]
        o_ref[...] = (
            jnp.dot(a, wl_ref[...], preferred_element_type=jnp.float32)
            + jnp.dot(x_ref[...], wr_ref[...], preferred_element_type=jnp.float32)
            + b_ref[...]
        )

    return pl.pallas_call(
        body,
        grid=(n // block_rows,),
        in_specs=[
            pl.BlockSpec((2, block_rows, d), lambda i: (0, i, 0)),
            pl.BlockSpec((block_rows, d), lambda i: (i, 0)),
            pl.BlockSpec((d, d), lambda i: (0, 0)),
            pl.BlockSpec((d, d), lambda i: (0, 0)),
            pl.BlockSpec((1, d), lambda i: (0, 0)),
        ],
        out_specs=pl.BlockSpec((block_rows, d), lambda i: (i, 0)),
        out_shape=jax.ShapeDtypeStruct((n, d), jnp.float32),
    )(partials, x, w_l, w_r, b_l)


def kernel(x, edge_index, edge_weight, W_l, b_l, W_r):
    n, d = x.shape
    e = edge_weight.shape[0]
    edges_per_tile = e // NW
    n_chunks = edges_per_tile // CHUNK

    src = edge_index[0].astype(jnp.int32).reshape(NW, n_chunks, CHUNK)
    dst = edge_index[1].astype(jnp.int32).reshape(NW, n_chunks, CHUNK)

    partials = _sc_aggregate(src, dst, x, edge_weight, n_chunks=n_chunks)
    return _tc_linear(partials, x, W_l, W_r, b_l.reshape(1, d))


# speedup vs baseline: 4.0078x; 4.0078x over previous
import functools

import jax
import jax.numpy as jnp
from jax import lax
from jax.experimental import pallas as pl
from jax.experimental.pallas import tpu as pltpu
from jax.experimental.pallas import tpu_sc as plsc

NC = 2
NS = 16
NW = NC * NS
LANES = 16

CHUNK = 80


def _sc_aggregate(src, dst, x, edge_weight, *, n_chunks):
    n, d = x.shape
    zrows = 80
    rows_per_tile = -(-n // NS)
    rows_per_tile += (-rows_per_tile) % zrows
    n_pad = NS * rows_per_tile
    n_stage = rows_per_tile // zrows

    mesh = plsc.VectorSubcoreMesh(
        core_axis_name="c", subcore_axis_name="s", num_cores=NC, num_subcores=NS
    )

    ib = 25
    n_iblocks = n_chunks // ib

    @functools.partial(
        pl.kernel,
        out_type=jax.ShapeDtypeStruct((NC, n_pad, d), jnp.float32),
        mesh=mesh,
        scratch_types=[
            pltpu.VMEM((ib, CHUNK), jnp.int32),
            pltpu.VMEM((ib, CHUNK), jnp.int32),
            pltpu.VMEM((CHUNK, d), jnp.float32),
            pltpu.VMEM((CHUNK, d), jnp.float32),
            pltpu.VMEM_SHARED((n_pad, d), jnp.float32),
            pltpu.SemaphoreType.DMA,
        ],
    )
    def agg_kernel(src_hbm, dst_hbm, x_hbm, ew_hbm, out_hbm,
                   src_v, dst_v, xbuf, ewbuf, agg_sh, sem):
        c = lax.axis_index("c")
        s = lax.axis_index("s")
        wid = s * NC + c
        edge_base = wid * (n_chunks * CHUNK)

        zero = jnp.zeros((LANES,), jnp.float32)

        def zero_row(r, _):
            for cc in range(d // LANES):
                ewbuf[r, pl.ds(cc * LANES, LANES)] = zero
            return 0

        lax.fori_loop(0, zrows, zero_row, 0)
        for k in range(n_stage):
            pltpu.sync_copy(
                ewbuf, agg_sh.at[pl.ds(s * rows_per_tile + k * zrows, zrows)]
            )
        plsc.subcore_barrier()

        def iblock_body(ob, _):
            pltpu.sync_copy(src_hbm.at[wid, ob], src_v)
            pltpu.sync_copy(dst_hbm.at[wid, ob], dst_v)

            def chunk_body(jj, _):
                j = ob * ib + jj
                pltpu.sync_copy(
                    ew_hbm.at[pl.ds(edge_base + j * CHUNK, CHUNK)], ewbuf
                )
                pltpu.async_copy(x_hbm.at[src_v.at[jj]], xbuf, sem).wait()

                def mul_row(r, _):
                    for cc in range(d // LANES):
                        sl = pl.ds(cc * LANES, LANES)
                        ewbuf[r, sl] = ewbuf[r, sl] * xbuf[r, sl]
                    return 0

                lax.fori_loop(0, CHUNK, mul_row, 0)
                pltpu.sync_copy(ewbuf, agg_sh.at[dst_v.at[jj]], add=True)
                return 0

            lax.fori_loop(0, ib, chunk_body, 0)
            return 0

        lax.fori_loop(0, n_iblocks, iblock_body, 0)
        plsc.subcore_barrier()

        for k in range(n_stage):
            rows = pl.ds(s * rows_per_tile + k * zrows, zrows)
            pltpu.sync_copy(agg_sh.at[rows], ewbuf)
            pltpu.sync_copy(ewbuf, out_hbm.at[c].at[rows])

    return agg_kernel(src, dst, x, edge_weight)


def _tc_linear(partials, x, w_l, w_r, b_l, *, block_rows=400):
    n, d = x.shape

    def body(p_ref, x_ref, wl_ref, wr_ref, b_ref, o_ref):
        a = p_ref[0] + p_ref[1]
        o_ref[...] = (
            jnp.dot(a, wl_ref[...], preferred_element_type=jnp.float32)
            + jnp.dot(x_ref[...], wr_ref[...], preferred_element_type=jnp.float32)
            + b_ref[...]
        )

    return pl.pallas_call(
        body,
        grid=(n // block_rows,),
        in_specs=[
            pl.BlockSpec((2, block_rows, d), lambda i: (0, i, 0)),
            pl.BlockSpec((block_rows, d), lambda i: (i, 0)),
            pl.BlockSpec((d, d), lambda i: (0, 0)),
            pl.BlockSpec((d, d), lambda i: (0, 0)),
            pl.BlockSpec((1, d), lambda i: (0, 0)),
        ],
        out_specs=pl.BlockSpec((block_rows, d), lambda i: (i, 0)),
        out_shape=jax.ShapeDtypeStruct((n, d), jnp.float32),
    )(partials, x, w_l, w_r, b_l)


def kernel(x, edge_index, edge_weight, W_l, b_l, W_r):
    n, d = x.shape
    e = edge_weight.shape[0]
    edges_per_tile = e // NW
    n_chunks = edges_per_tile // CHUNK

    src = edge_index[0].astype(jnp.int32).reshape(NW, n_chunks // 25, 25, CHUNK)
    dst = edge_index[1].astype(jnp.int32).reshape(NW, n_chunks // 25, 25, CHUNK)

    partials = _sc_aggregate(src, dst, x, edge_weight, n_chunks=n_chunks)
    return _tc_linear(partials, x, W_l, W_r, b_l.reshape(1, d))


# speedup vs baseline: 6.7379x; 1.6812x over previous
import functools

import jax
import jax.numpy as jnp
from jax import lax
from jax.experimental import pallas as pl
from jax.experimental.pallas import tpu as pltpu
from jax.experimental.pallas import tpu_sc as plsc

NC = 2
NS = 16
NW = NC * NS
LANES = 16

CHUNK = 40
IB = 50


def _sc_aggregate(src, dst, x, edge_weight, *, n_chunks):
    n, d = x.shape
    rows_per_tile = -(-n // NS)
    rows_per_tile += (-rows_per_tile) % CHUNK
    n_pad = NS * rows_per_tile
    n_stage = rows_per_tile // CHUNK
    n_iblocks = n_chunks // IB
    pairs = IB // 2

    mesh = plsc.VectorSubcoreMesh(
        core_axis_name="c", subcore_axis_name="s", num_cores=NC, num_subcores=NS
    )

    @functools.partial(
        pl.kernel,
        out_type=jax.ShapeDtypeStruct((NC, n_pad, d), jnp.float32),
        mesh=mesh,
        scratch_types=[
            pltpu.VMEM((IB, CHUNK), jnp.int32),
            pltpu.VMEM((IB, CHUNK), jnp.int32),
            pltpu.VMEM((CHUNK, d), jnp.float32),
            pltpu.VMEM((CHUNK, d), jnp.float32),
            pltpu.VMEM((CHUNK, d), jnp.float32),
            pltpu.VMEM((CHUNK, d), jnp.float32),
            pltpu.VMEM_SHARED((n_pad, d), jnp.float32),
            pltpu.SemaphoreType.DMA,
            pltpu.SemaphoreType.DMA,
            pltpu.SemaphoreType.DMA,
            pltpu.SemaphoreType.DMA,
            pltpu.SemaphoreType.DMA,
            pltpu.SemaphoreType.DMA,
        ],
    )
    def agg_kernel(src_hbm, dst_hbm, x_hbm, ew_hbm, out_hbm,
                   src_v, dst_v, xb0, xb1, eb0, eb1, agg_sh,
                   sx0, sx1, se0, se1, ss0, ss1):
        c = lax.axis_index("c")
        s = lax.axis_index("s")
        wid = s * NC + c
        edge_base = wid * (n_chunks * CHUNK)
        xbufs, ebufs = [xb0, xb1], [eb0, eb1]
        sxs, ses, sss = [sx0, sx1], [se0, se1], [ss0, ss1]

        zero = jnp.zeros((LANES,), jnp.float32)

        def zero_row(r, _):
            for cc in range(d // LANES):
                eb0[r, pl.ds(cc * LANES, LANES)] = zero
            return 0

        lax.fori_loop(0, CHUNK, zero_row, 0)
        for k in range(n_stage):
            pltpu.sync_copy(
                eb0, agg_sh.at[pl.ds(s * rows_per_tile + k * CHUNK, CHUNK)]
            )
        plsc.subcore_barrier()

        def ew_desc(ob, jj, p):
            off = edge_base + (ob * IB + jj) * CHUNK
            return pltpu.make_async_copy(
                ew_hbm.at[pl.ds(off, CHUNK)], ebufs[p], ses[p]
            )

        def gather_desc(jj, p):
            return pltpu.make_async_copy(
                x_hbm.at[src_v.at[jj]], xbufs[p], sxs[p]
            )

        def issue_in(ob, jj, p):
            ew_desc(ob, jj, p).start()
            gather_desc(jj, p).start()

        def issue_scatter(jj, p):
            pltpu.async_copy(ebufs[p], agg_sh.at[dst_v.at[jj]], sss[p], add=True)

        def wait_scatter(jj, p):
            pltpu.make_async_copy(ebufs[p], agg_sh.at[dst_v.at[jj]], sss[p]).wait()

        def compute(jj, p):
            def mul_row(r, _):
                for cc in range(d // LANES):
                    sl = pl.ds(cc * LANES, LANES)
                    ebufs[p][r, sl] = ebufs[p][r, sl] * xbufs[p][r, sl]
                return 0

            lax.fori_loop(0, CHUNK, mul_row, 0)

        def block_body(ob, _):
            pltpu.sync_copy(src_hbm.at[wid, ob], src_v)
            pltpu.sync_copy(dst_hbm.at[wid, ob], dst_v)
            issue_in(ob, 0, 0)

            def pair_body(pr, _):
                for b in (0, 1):
                    jj = pr * 2 + b
                    p, q = b, 1 - b
                    if b == 0:
                        @pl.when(pr > 0)
                        def _():
                            wait_scatter(jj - 1, q)
                        issue_in(ob, jj + 1, q)
                    else:
                        wait_scatter(jj - 1, q)

                        @pl.when(pr < pairs - 1)
                        def _():
                            issue_in(ob, jj + 1, q)
                    ew_desc(ob, jj, p).wait()
                    gather_desc(jj, p).wait()
                    compute(jj, p)
                    issue_scatter(jj, p)
                return 0

            lax.fori_loop(0, pairs, pair_body, 0)
            wait_scatter(IB - 1, 1)
            return 0

        lax.fori_loop(0, n_iblocks, block_body, 0)
        plsc.subcore_barrier()

        for k in range(n_stage):
            rows = pl.ds(s * rows_per_tile + k * CHUNK, CHUNK)
            pltpu.sync_copy(agg_sh.at[rows], eb0)
            pltpu.sync_copy(eb0, out_hbm.at[c].at[rows])

    return agg_kernel(src, dst, x, edge_weight)


def _tc_linear(partials, x, w_l, w_r, b_l, *, block_rows=400):
    n, d = x.shape

    def body(p_ref, x_ref, wl_ref, wr_ref, b_ref, o_ref):
        a = p_ref[0] + p_ref[1]
        o_ref[...] = (
            jnp.dot(a, wl_ref[...], preferred_element_type=jnp.float32)
            + jnp.dot(x_ref[...], wr_ref[...], preferred_element_type=jnp.float32)
            + b_ref[...]
        )

    return pl.pallas_call(
        body,
        grid=(n // block_rows,),
        in_specs=[
            pl.BlockSpec((2, block_rows, d), lambda i: (0, i, 0)),
            pl.BlockSpec((block_rows, d), lambda i: (i, 0)),
            pl.BlockSpec((d, d), lambda i: (0, 0)),
            pl.BlockSpec((d, d), lambda i: (0, 0)),
            pl.BlockSpec((1, d), lambda i: (0, 0)),
        ],
        out_specs=pl.BlockSpec((block_rows, d), lambda i: (i, 0)),
        out_shape=jax.ShapeDtypeStruct((n, d), jnp.float32),
    )(partials, x, w_l, w_r, b_l)


def kernel(x, edge_index, edge_weight, W_l, b_l, W_r):
    n, d = x.shape
    e = edge_weight.shape[0]
    edges_per_tile = e // NW
    n_chunks = edges_per_tile // CHUNK

    src = edge_index[0].astype(jnp.int32).reshape(NW, n_chunks // IB, IB, CHUNK)
    dst = edge_index[1].astype(jnp.int32).reshape(NW, n_chunks // IB, IB, CHUNK)

    partials = _sc_aggregate(src, dst, x, edge_weight, n_chunks=n_chunks)
    return _tc_linear(partials, x, W_l, W_r, b_l.reshape(1, d))
